# baseline (speedup 1.0000x reference)
import functools

import jax
import jax.numpy as jnp
from jax.experimental import pallas as pl

_ROWS_PER_BLOCK = 512


def _asnc_ln_body(t_ref, dy_ref, y0_ref, gamma_ref, beta_ref, x_ref, o_ref,
                  *, n_thresh):
    x = x_ref[...]
    acc = jnp.broadcast_to(y0_ref[...], x.shape)
    for j in range(n_thresh):
        tj = t_ref[j:j + 1, :]
        dj = dy_ref[j:j + 1, :]
        acc = acc + jnp.where(x > tj, dj, jnp.float32(0.0))
    mean = jnp.mean(acc, axis=-1, keepdims=True)
    cen = acc - mean
    var = jnp.mean(cen * cen, axis=-1, keepdims=True)
    inv = jax.lax.rsqrt(var + jnp.float32(1e-5))
    o_ref[...] = cen * inv * gamma_ref[...] + beta_ref[...]


@jax.jit
def kernel(x, thresholds, y, gamma, beta):
    shape = x.shape
    H = shape[-1]
    Km1 = thresholds.shape[1]
    x2 = x.reshape(-1, H)
    N = x2.shape[0]

    t_t = thresholds.T
    dy_t = (y[:, 1:] - y[:, :-1]).T
    y0 = y[:, 0].reshape(1, H)
    gamma2 = gamma.reshape(1, H)
    beta2 = beta.reshape(1, H)

    R = _ROWS_PER_BLOCK
    grid = (N // R,)

    out = pl.pallas_call(
        functools.partial(_asnc_ln_body, n_thresh=Km1),
        grid=grid,
        in_specs=[
            pl.BlockSpec((Km1, H), lambda i: (0, 0)),
            pl.BlockSpec((Km1, H), lambda i: (0, 0)),
            pl.BlockSpec((1, H), lambda i: (0, 0)),
            pl.BlockSpec((1, H), lambda i: (0, 0)),
            pl.BlockSpec((1, H), lambda i: (0, 0)),
            pl.BlockSpec((R, H), lambda i: (i, 0)),
        ],
        out_specs=pl.BlockSpec((R, H), lambda i: (i, 0)),
        out_shape=jax.ShapeDtypeStruct((N, H), x.dtype),
    )(t_t, dy_t, y0, gamma2, beta2, x2)
    return out.reshape(shape)


# speedup vs baseline: 1.2882x; 1.2882x over previous
import functools

import jax
import jax.numpy as jnp
from jax.experimental import pallas as pl

_ROWS_PER_BLOCK = 128


def _asnc_ln_body(t_ref, dy_ref, y0_ref, gamma_ref, beta_ref, x_ref, o_ref,
                  *, n_thresh):
    x = x_ref[...]
    acc = jnp.broadcast_to(y0_ref[...], x.shape)
    for j in range(n_thresh):
        tj = t_ref[j:j + 1, :]
        dj = dy_ref[j:j + 1, :]
        acc = acc + jnp.where(x > tj, dj, jnp.float32(0.0))
    mean = jnp.mean(acc, axis=-1, keepdims=True)
    cen = acc - mean
    var = jnp.mean(cen * cen, axis=-1, keepdims=True)
    inv = jax.lax.rsqrt(var + jnp.float32(1e-5))
    o_ref[...] = cen * inv * gamma_ref[...] + beta_ref[...]


@jax.jit
def kernel(x, thresholds, y, gamma, beta):
    shape = x.shape
    H = shape[-1]
    Km1 = thresholds.shape[1]
    x2 = x.reshape(-1, H)
    N = x2.shape[0]

    t_t = thresholds.T
    dy_t = (y[:, 1:] - y[:, :-1]).T
    y0 = y[:, 0].reshape(1, H)
    gamma2 = gamma.reshape(1, H)
    beta2 = beta.reshape(1, H)

    R = _ROWS_PER_BLOCK
    grid = (N // R,)

    out = pl.pallas_call(
        functools.partial(_asnc_ln_body, n_thresh=Km1),
        grid=grid,
        in_specs=[
            pl.BlockSpec((Km1, H), lambda i: (0, 0)),
            pl.BlockSpec((Km1, H), lambda i: (0, 0)),
            pl.BlockSpec((1, H), lambda i: (0, 0)),
            pl.BlockSpec((1, H), lambda i: (0, 0)),
            pl.BlockSpec((1, H), lambda i: (0, 0)),
            pl.BlockSpec((R, H), lambda i: (i, 0)),
        ],
        out_specs=pl.BlockSpec((R, H), lambda i: (i, 0)),
        out_shape=jax.ShapeDtypeStruct((N, H), x.dtype),
    )(t_t, dy_t, y0, gamma2, beta2, x2)
    return out.reshape(shape)


# speedup vs baseline: 1.3333x; 1.0350x over previous
import functools

import jax
import jax.numpy as jnp
from jax.experimental import pallas as pl

_ROWS_PER_BLOCK = 256
_SUB = 8


def _asnc_ln_body(t_ref, dy_ref, y0c_ref, gamma_ref, beta_ref, x_ref, o_ref,
                  *, n_thresh):
    x = x_ref[...]
    acc = jnp.where(x > t_ref[0:1], dy_ref[0:1], jnp.float32(0.0))
    for j in range(1, n_thresh):
        acc = acc + jnp.where(x > t_ref[j:j + 1], dy_ref[j:j + 1],
                              jnp.float32(0.0))
    m = jnp.mean(acc, axis=-1, keepdims=True)
    cen = (acc - m) + y0c_ref[...]
    var = jnp.mean(cen * cen, axis=-1, keepdims=True)
    inv = jax.lax.rsqrt(var + jnp.float32(1e-5))
    o_ref[...] = cen * inv * gamma_ref[...] + beta_ref[...]


@jax.jit
def kernel(x, thresholds, y, gamma, beta):
    shape = x.shape
    H = shape[-1]
    Km1 = thresholds.shape[1]
    x3 = x.reshape(-1, _SUB, H)
    G = x3.shape[0]

    t8 = jnp.broadcast_to(thresholds.T[:, None, :], (Km1, _SUB, H))
    dy8 = jnp.broadcast_to((y[:, 1:] - y[:, :-1]).T[:, None, :],
                           (Km1, _SUB, H))
    y0 = y[:, 0]
    y0c = jnp.broadcast_to((y0 - jnp.mean(y0))[None, None, :], (1, _SUB, H))
    gamma3 = jnp.broadcast_to(gamma[None, None, :], (1, _SUB, H))
    beta3 = jnp.broadcast_to(beta[None, None, :], (1, _SUB, H))

    Rg = _ROWS_PER_BLOCK // _SUB
    grid = (G // Rg,)

    out = pl.pallas_call(
        functools.partial(_asnc_ln_body, n_thresh=Km1),
        grid=grid,
        in_specs=[
            pl.BlockSpec((Km1, _SUB, H), lambda i: (0, 0, 0)),
            pl.BlockSpec((Km1, _SUB, H), lambda i: (0, 0, 0)),
            pl.BlockSpec((1, _SUB, H), lambda i: (0, 0, 0)),
            pl.BlockSpec((1, _SUB, H), lambda i: (0, 0, 0)),
            pl.BlockSpec((1, _SUB, H), lambda i: (0, 0, 0)),
            pl.BlockSpec((Rg, _SUB, H), lambda i: (i, 0, 0)),
        ],
        out_specs=pl.BlockSpec((Rg, _SUB, H), lambda i: (i, 0, 0)),
        out_shape=jax.ShapeDtypeStruct((G, _SUB, H), x.dtype),
    )(t8, dy8, y0c, gamma3, beta3, x3)
    return out.reshape(shape)


# speedup vs baseline: 1.5580x; 1.1686x over previous
import functools

import jax
import jax.numpy as jnp
from jax.experimental import pallas as pl

_ROWS_PER_BLOCK = 256
_SUB = 8


def _asnc_ln_body(t_ref, y_ref, gamma_ref, beta_ref, x_ref, o_ref,
                  *, n_thresh):
    x = x_ref[...]
    v = jnp.where(x > t_ref[0:1], y_ref[1:2], y_ref[0:1])
    for j in range(1, n_thresh):
        v = jnp.where(x > t_ref[j:j + 1], y_ref[j + 1:j + 2], v)
    m = jnp.mean(v, axis=-1, keepdims=True)
    cen = v - m
    var = jnp.mean(cen * cen, axis=-1, keepdims=True)
    inv = jax.lax.rsqrt(var + jnp.float32(1e-5))
    o_ref[...] = cen * inv * gamma_ref[...] + beta_ref[...]


@jax.jit
def kernel(x, thresholds, y, gamma, beta):
    shape = x.shape
    H = shape[-1]
    Km1 = thresholds.shape[1]
    K = y.shape[1]
    x3 = x.reshape(-1, _SUB, H)
    G = x3.shape[0]

    t8 = jnp.broadcast_to(thresholds.T[:, None, :], (Km1, _SUB, H))
    y8 = jnp.broadcast_to(y.T[:, None, :], (K, _SUB, H))
    gamma3 = jnp.broadcast_to(gamma[None, None, :], (1, _SUB, H))
    beta3 = jnp.broadcast_to(beta[None, None, :], (1, _SUB, H))

    Rg = _ROWS_PER_BLOCK // _SUB
    grid = (G // Rg,)

    out = pl.pallas_call(
        functools.partial(_asnc_ln_body, n_thresh=Km1),
        grid=grid,
        in_specs=[
            pl.BlockSpec((Km1, _SUB, H), lambda i: (0, 0, 0)),
            pl.BlockSpec((K, _SUB, H), lambda i: (0, 0, 0)),
            pl.BlockSpec((1, _SUB, H), lambda i: (0, 0, 0)),
            pl.BlockSpec((1, _SUB, H), lambda i: (0, 0, 0)),
            pl.BlockSpec((Rg, _SUB, H), lambda i: (i, 0, 0)),
        ],
        out_specs=pl.BlockSpec((Rg, _SUB, H), lambda i: (i, 0, 0)),
        out_shape=jax.ShapeDtypeStruct((G, _SUB, H), x.dtype),
    )(t8, y8, gamma3, beta3, x3)
    return out.reshape(shape)


# speedup vs baseline: 1.6974x; 1.0894x over previous
import functools

import jax
import jax.numpy as jnp
from jax.experimental import pallas as pl

_ROWS_PER_BLOCK = 256
_SUB = 8
_LANES = 128


def _asnc_ln_body(t_ref, y_ref, gamma_ref, beta_ref, x_ref, o_ref,
                  *, n_thresh, h):
    Rg = x_ref.shape[0]
    n_chunks = h // _LANES
    s = jnp.zeros((Rg, _SUB, _LANES), jnp.float32)
    s2 = jnp.zeros((Rg, _SUB, _LANES), jnp.float32)
    for c in range(n_chunks):
        sl = slice(c * _LANES, (c + 1) * _LANES)
        xc = x_ref[:, :, sl]
        v = jnp.where(xc > t_ref[0:1, :, sl], y_ref[1:2, :, sl],
                      y_ref[0:1, :, sl])
        for j in range(1, n_thresh):
            v = jnp.where(xc > t_ref[j:j + 1, :, sl],
                          y_ref[j + 1:j + 2, :, sl], v)
        o_ref[:, :, sl] = v
        s = s + v
        s2 = s2 + v * v
    m = jnp.sum(s, axis=-1, keepdims=True) * (1.0 / h)
    ex2 = jnp.sum(s2, axis=-1, keepdims=True) * (1.0 / h)
    var = ex2 - m * m
    inv = jax.lax.rsqrt(var + jnp.float32(1e-5))
    for c in range(n_chunks):
        sl = slice(c * _LANES, (c + 1) * _LANES)
        v = o_ref[:, :, sl]
        o_ref[:, :, sl] = ((v - m) * inv * gamma_ref[0:1, :, sl]
                           + beta_ref[0:1, :, sl])


@jax.jit
def kernel(x, thresholds, y, gamma, beta):
    shape = x.shape
    H = shape[-1]
    Km1 = thresholds.shape[1]
    K = y.shape[1]
    x3 = x.reshape(-1, _SUB, H)
    G = x3.shape[0]

    t8 = jnp.broadcast_to(thresholds.T[:, None, :], (Km1, _SUB, H))
    y8 = jnp.broadcast_to(y.T[:, None, :], (K, _SUB, H))
    gamma3 = jnp.broadcast_to(gamma[None, None, :], (1, _SUB, H))
    beta3 = jnp.broadcast_to(beta[None, None, :], (1, _SUB, H))

    Rg = _ROWS_PER_BLOCK // _SUB
    grid = (G // Rg,)

    out = pl.pallas_call(
        functools.partial(_asnc_ln_body, n_thresh=Km1, h=H),
        grid=grid,
        in_specs=[
            pl.BlockSpec((Km1, _SUB, H), lambda i: (0, 0, 0)),
            pl.BlockSpec((K, _SUB, H), lambda i: (0, 0, 0)),
            pl.BlockSpec((1, _SUB, H), lambda i: (0, 0, 0)),
            pl.BlockSpec((1, _SUB, H), lambda i: (0, 0, 0)),
            pl.BlockSpec((Rg, _SUB, H), lambda i: (i, 0, 0)),
        ],
        out_specs=pl.BlockSpec((Rg, _SUB, H), lambda i: (i, 0, 0)),
        out_shape=jax.ShapeDtypeStruct((G, _SUB, H), x.dtype),
    )(t8, y8, gamma3, beta3, x3)
    return out.reshape(shape)
